# baseline (speedup 1.0000x reference)
import numpy as np

import jax
import jax.numpy as jnp
from jax import lax
from jax.experimental import pallas as pl
from jax.experimental.pallas import tpu as pltpu
from jax.experimental.pallas import tpu_sc as plsc

N = 10000
E = 320000
D = 128
DH = D // 2
NC = 2
NS = 16
EPT = E // NS
CH = 80
G = 5
NG = EPT // (CH * G)
RPS = 624
TAIL = N - NS * RPS
ZB = 78

_PERM64 = np.concatenate([np.arange(0, 32, 2), np.arange(1, 32, 2),
                          np.arange(32, 64, 2), np.arange(33, 64, 2)])
_PERM = np.concatenate([_PERM64, DH + _PERM64])


def _scale_rows(st, rows_bf, w_blk, b):
    hi_mask = jnp.full((16,), -65536, jnp.int32)

    @plsc.parallel_loop(0, CH, 16)
    def body(r0):
        w16 = w_blk[b, pl.ds(r0, 16)]
        for i in range(16):
            wb = jnp.full((16,), w16[i], jnp.float32)
            r = r0 + i
            for v in range(DH // 32):
                xb = rows_bf[b, r, pl.ds(v * 32, 32)]
                xi = plsc.bitcast(xb, jnp.int32)
                st[b, r, pl.ds(v * 32, 16)] = (
                    plsc.bitcast(xi << 16, jnp.float32) * wb)
                st[b, r, pl.ds(v * 32 + 16, 16)] = (
                    plsc.bitcast(xi & hi_mask, jnp.float32) * wb)


def _spmm_body(x_hbm, src_hbm, dst_hbm, w_hbm, out_hbm,
               srcA, dstA, wA, srcB, dstB, wB, rows_bf, st, zbuf, h_sh,
               gsems, ssems, isemA, isemB):
    c = lax.axis_index("c")
    s = lax.axis_index("s")

    def load_idx_block(g, bufs, sem):
        return [pltpu.async_copy(src_hbm.at[s, g], bufs[0], sem),
                pltpu.async_copy(dst_hbm.at[s, g], bufs[1], sem),
                pltpu.async_copy(w_hbm.at[s, g], bufs[2], sem)]

    def wait_idx_block(bufs, sem):
        for src_r, buf in zip((src_hbm.at[s, 0], dst_hbm.at[s, 0],
                               w_hbm.at[s, 0]), bufs):
            pltpu.make_async_copy(src_r, buf, sem).wait()

    def gather(src_blk, b):
        return pltpu.async_copy(x_hbm.at[c].at[src_blk.at[b]], rows_bf.at[b],
                                gsems[b])

    def wait_gather(src_blk, b):
        pltpu.make_async_copy(x_hbm.at[c].at[src_blk.at[b]], rows_bf.at[b],
                              gsems[b]).wait()

    def scatter(dst_blk, b):
        return pltpu.async_copy(st.at[b], h_sh.at[dst_blk.at[b]], ssems[b],
                                add=True)

    def wait_scatter(dst_blk, b):
        pltpu.make_async_copy(st.at[b], h_sh.at[dst_blk.at[b]],
                              ssems[b]).wait()

    for d in load_idx_block(0, (srcA, dstA, wA), isemA):
        d.wait()
    for b in range(G):
        gather(srcA, b)

    zero16 = jnp.zeros((16,), jnp.float32)

    def _zrow(i, carry):
        for v in range(DH // 16):
            zbuf[i, pl.ds(v * 16, 16)] = zero16
        return carry

    lax.fori_loop(0, ZB, _zrow, 0)
    row0 = s * RPS
    for k in range(RPS // ZB):
        pltpu.sync_copy(zbuf, h_sh.at[pl.ds(row0 + k * ZB, ZB)])

    @pl.when(s == NS - 1)
    def _zero_tail():
        pltpu.sync_copy(zbuf.at[pl.ds(0, TAIL)],
                        h_sh.at[pl.ds(NS * RPS, TAIL)])

    plsc.subcore_barrier()

    def _pair(m, carry):
        g0 = 2 * m
        load_idx_block(g0 + 1, (srcB, dstB, wB), isemB)
        for b in range(G):
            wait_gather(srcA, b)
            _scale_rows(st, rows_bf, wA, b)
            scatter(dstA, b)
        wait_idx_block((srcB, dstB, wB), isemB)
        for b in range(G):
            wait_scatter(dstA, b)
            gather(srcB, b)

        @pl.when(m < NG // 2 - 1)
        def _prefetch_a():
            load_idx_block(g0 + 2, (srcA, dstA, wA), isemA)

        for b in range(G):
            wait_gather(srcB, b)
            _scale_rows(st, rows_bf, wB, b)
            scatter(dstB, b)

        @pl.when(m < NG // 2 - 1)
        def _next_gathers():
            wait_idx_block((srcA, dstA, wA), isemA)
            for b in range(G):
                wait_scatter(dstB, b)
                gather(srcA, b)

        return carry

    lax.fori_loop(0, NG // 2, _pair, 0)
    for b in range(G):
        wait_scatter(dstB, b)
    plsc.subcore_barrier()

    pltpu.sync_copy(h_sh.at[pl.ds(row0, RPS)],
                    out_hbm.at[c].at[pl.ds(row0, RPS)])

    @pl.when(s == NS - 1)
    def _write_tail():
        pltpu.sync_copy(h_sh.at[pl.ds(NS * RPS, TAIL)],
                        out_hbm.at[c].at[pl.ds(NS * RPS, TAIL)])


def _spmm(x2, src4, dst4, w4):
    mesh = plsc.VectorSubcoreMesh(core_axis_name="c", subcore_axis_name="s")

    def body(x_hbm, src_hbm, dst_hbm, w_hbm, out_hbm, srcA, dstA, wA,
             srcB, dstB, wB, rows_bf, st, zbuf, h_sh,
             g0, g1, g2, g3, g4, s0, s1, s2, s3, s4, iA, iB):
        _spmm_body(x_hbm, src_hbm, dst_hbm, w_hbm, out_hbm, srcA, dstA, wA,
                   srcB, dstB, wB, rows_bf, st, zbuf, h_sh,
                   [g0, g1, g2, g3, g4], [s0, s1, s2, s3, s4], iA, iB)

    f = pl.kernel(
        body,
        out_type=jax.ShapeDtypeStruct((NC, N, DH), jnp.float32),
        mesh=mesh,
        scratch_types=[
            pltpu.VMEM((G, CH), jnp.int32),
            pltpu.VMEM((G, CH), jnp.int32),
            pltpu.VMEM((G, CH), jnp.float32),
            pltpu.VMEM((G, CH), jnp.int32),
            pltpu.VMEM((G, CH), jnp.int32),
            pltpu.VMEM((G, CH), jnp.float32),
            pltpu.VMEM((G, CH, DH), jnp.bfloat16),
            pltpu.VMEM((G, CH, DH), jnp.float32),
            pltpu.VMEM((ZB, DH), jnp.float32),
            pltpu.VMEM_SHARED((N, DH), jnp.float32),
        ] + [pltpu.SemaphoreType.DMA] * 12,
        compiler_params=pltpu.CompilerParams(use_tc_tiling_on_sc=False,
                                             needs_layout_passes=False),
    )
    return f(x2, src4, dst4, w4)


def _linear_body(h0_ref, h1_ref, w_ref, b_ref, o_ref):
    h = jnp.concatenate([h0_ref[0], h1_ref[0]], axis=1)
    acc = jnp.dot(h, w_ref[...], preferred_element_type=jnp.float32)
    o_ref[...] = jnp.maximum(acc + b_ref[...], 0.0)


def _linear(h2, Wp, b):
    blk = 1000
    grid = (N // blk,)
    return pl.pallas_call(
        _linear_body,
        grid=grid,
        in_specs=[
            pl.BlockSpec((1, blk, DH), lambda i: (0, i, 0)),
            pl.BlockSpec((1, blk, DH), lambda i: (1, i, 0)),
            pl.BlockSpec((D, D), lambda i: (0, 0)),
            pl.BlockSpec((1, D), lambda i: (0, 0)),
        ],
        out_specs=pl.BlockSpec((blk, D), lambda i: (i, 0)),
        out_shape=jax.ShapeDtypeStruct((N, D), jnp.float32),
    )(h2, h2, Wp, b.reshape(1, D))


def kernel(x, edge_index, edge_weight, W, b):
    x2 = jnp.stack([x[:, :DH], x[:, DH:]]).astype(jnp.bfloat16)
    src4 = edge_index[0].reshape(NS, NG, G, CH)
    dst4 = edge_index[1].reshape(NS, NG, G, CH)
    w4 = edge_weight.reshape(NS, NG, G, CH)
    h2 = _spmm(x2, src4, dst4, w4)
    return _linear(h2, W[_PERM, :], b)


# speedup vs baseline: 1.1212x; 1.1212x over previous
import numpy as np

import jax
import jax.numpy as jnp
from jax import lax
from jax.experimental import pallas as pl
from jax.experimental.pallas import tpu as pltpu
from jax.experimental.pallas import tpu_sc as plsc

N = 10000
E = 320000
D = 128
DH = D // 2
NC = 2
NS = 16
EPT = E // NS
CH = 80
G = 5
NG = EPT // (CH * G)
RPS = 624
TAIL = N - NS * RPS
ZB = 78

_PERM64 = np.concatenate([np.arange(0, 32, 2), np.arange(1, 32, 2),
                          np.arange(32, 64, 2), np.arange(33, 64, 2)])
_PERM = np.concatenate([_PERM64, DH + _PERM64])


def _scale_rows(st, rows_bf, w_blk, b):
    hi_mask = jnp.full((16,), -65536, jnp.int32)

    @plsc.parallel_loop(0, CH, unroll=8)
    def body(r):
        wb = w_blk[b, r]
        for v in range(DH // 32):
            xb = rows_bf[b, r, pl.ds(v * 32, 32)]
            xi = plsc.bitcast(xb, jnp.int32)
            ev = plsc.bitcast(xi << 16, jnp.float32)
            od = plsc.bitcast(xi & hi_mask, jnp.float32)
            st[b, r, pl.ds(v * 32, 16)] = ev * wb
            st[b, r, pl.ds(v * 32 + 16, 16)] = od * wb


def _spmm_body(x_hbm, src_hbm, dst_hbm, w_hbm, out_hbm,
               srcA, dstA, wA, srcB, dstB, wB, rows_bf, st, zbuf, h_sh,
               gsems, ssems, isemA, isemB):
    c = lax.axis_index("c")
    s = lax.axis_index("s")

    def load_idx_block(g, bufs, sem):
        return [pltpu.async_copy(src_hbm.at[s, g], bufs[0], sem),
                pltpu.async_copy(dst_hbm.at[s, g], bufs[1], sem),
                pltpu.async_copy(w_hbm.at[s, g], bufs[2], sem)]

    def wait_idx_block(bufs, sem):
        for src_r, buf in zip((src_hbm.at[s, 0], dst_hbm.at[s, 0],
                               w_hbm.at[s, 0]), bufs):
            pltpu.make_async_copy(src_r, buf, sem).wait()

    def gather(src_blk, b):
        return pltpu.async_copy(x_hbm.at[c].at[src_blk.at[b]], rows_bf.at[b],
                                gsems[b])

    def wait_gather(src_blk, b):
        pltpu.make_async_copy(x_hbm.at[c].at[src_blk.at[b]], rows_bf.at[b],
                              gsems[b]).wait()

    def scatter(dst_blk, b):
        return pltpu.async_copy(st.at[b], h_sh.at[dst_blk.at[b]], ssems[b],
                                add=True)

    def wait_scatter(dst_blk, b):
        pltpu.make_async_copy(st.at[b], h_sh.at[dst_blk.at[b]],
                              ssems[b]).wait()

    for d in load_idx_block(0, (srcA, dstA, wA), isemA):
        d.wait()
    for b in range(G):
        gather(srcA, b)

    zero16 = jnp.zeros((16,), jnp.float32)

    def _zrow(i, carry):
        for v in range(DH // 16):
            zbuf[i, pl.ds(v * 16, 16)] = zero16
        return carry

    lax.fori_loop(0, ZB, _zrow, 0)
    row0 = s * RPS
    for k in range(RPS // ZB):
        pltpu.sync_copy(zbuf, h_sh.at[pl.ds(row0 + k * ZB, ZB)])

    @pl.when(s == NS - 1)
    def _zero_tail():
        pltpu.sync_copy(zbuf.at[pl.ds(0, TAIL)],
                        h_sh.at[pl.ds(NS * RPS, TAIL)])

    plsc.subcore_barrier()

    def _pair(m, carry):
        g0 = 2 * m
        load_idx_block(g0 + 1, (srcB, dstB, wB), isemB)
        for b in range(G):
            wait_gather(srcA, b)
            _scale_rows(st, rows_bf, wA, b)
            scatter(dstA, b)
        wait_idx_block((srcB, dstB, wB), isemB)
        for b in range(G):
            wait_scatter(dstA, b)
            gather(srcB, b)

        @pl.when(m < NG // 2 - 1)
        def _prefetch_a():
            load_idx_block(g0 + 2, (srcA, dstA, wA), isemA)

        for b in range(G):
            wait_gather(srcB, b)
            _scale_rows(st, rows_bf, wB, b)
            scatter(dstB, b)

        @pl.when(m < NG // 2 - 1)
        def _next_gathers():
            wait_idx_block((srcA, dstA, wA), isemA)
            for b in range(G):
                wait_scatter(dstB, b)
                gather(srcA, b)

        return carry

    lax.fori_loop(0, NG // 2, _pair, 0)
    for b in range(G):
        wait_scatter(dstB, b)
    plsc.subcore_barrier()

    pltpu.sync_copy(h_sh.at[pl.ds(row0, RPS)],
                    out_hbm.at[c].at[pl.ds(row0, RPS)])

    @pl.when(s == NS - 1)
    def _write_tail():
        pltpu.sync_copy(h_sh.at[pl.ds(NS * RPS, TAIL)],
                        out_hbm.at[c].at[pl.ds(NS * RPS, TAIL)])


def _spmm(x2, src4, dst4, w4):
    mesh = plsc.VectorSubcoreMesh(core_axis_name="c", subcore_axis_name="s")

    def body(x_hbm, src_hbm, dst_hbm, w_hbm, out_hbm, srcA, dstA, wA,
             srcB, dstB, wB, rows_bf, st, zbuf, h_sh,
             g0, g1, g2, g3, g4, s0, s1, s2, s3, s4, iA, iB):
        _spmm_body(x_hbm, src_hbm, dst_hbm, w_hbm, out_hbm, srcA, dstA, wA,
                   srcB, dstB, wB, rows_bf, st, zbuf, h_sh,
                   [g0, g1, g2, g3, g4], [s0, s1, s2, s3, s4], iA, iB)

    f = pl.kernel(
        body,
        out_type=jax.ShapeDtypeStruct((NC, N, DH), jnp.float32),
        mesh=mesh,
        scratch_types=[
            pltpu.VMEM((G, CH), jnp.int32),
            pltpu.VMEM((G, CH), jnp.int32),
            pltpu.VMEM((G, CH, 16), jnp.float32),
            pltpu.VMEM((G, CH), jnp.int32),
            pltpu.VMEM((G, CH), jnp.int32),
            pltpu.VMEM((G, CH, 16), jnp.float32),
            pltpu.VMEM((G, CH, DH), jnp.bfloat16),
            pltpu.VMEM((G, CH, DH), jnp.float32),
            pltpu.VMEM((ZB, DH), jnp.float32),
            pltpu.VMEM_SHARED((N, DH), jnp.float32),
        ] + [pltpu.SemaphoreType.DMA] * 12,
        compiler_params=pltpu.CompilerParams(use_tc_tiling_on_sc=False,
                                             needs_layout_passes=False),
    )
    return f(x2, src4, dst4, w4)


def _linear_body(h0_ref, h1_ref, w_ref, b_ref, o_ref):
    h = jnp.concatenate([h0_ref[0], h1_ref[0]], axis=1)
    acc = jnp.dot(h, w_ref[...], preferred_element_type=jnp.float32)
    o_ref[...] = jnp.maximum(acc + b_ref[...], 0.0)


def _linear(h2, Wp, b):
    blk = 1000
    grid = (N // blk,)
    return pl.pallas_call(
        _linear_body,
        grid=grid,
        in_specs=[
            pl.BlockSpec((1, blk, DH), lambda i: (0, i, 0)),
            pl.BlockSpec((1, blk, DH), lambda i: (1, i, 0)),
            pl.BlockSpec((D, D), lambda i: (0, 0)),
            pl.BlockSpec((1, D), lambda i: (0, 0)),
        ],
        out_specs=pl.BlockSpec((blk, D), lambda i: (i, 0)),
        out_shape=jax.ShapeDtypeStruct((N, D), jnp.float32),
    )(h2, h2, Wp, b.reshape(1, D))


def kernel(x, edge_index, edge_weight, W, b):
    x2 = jnp.stack([x[:, :DH], x[:, DH:]]).astype(jnp.bfloat16)
    src4 = edge_index[0].reshape(NS, NG, G, CH)
    dst4 = edge_index[1].reshape(NS, NG, G, CH)
    w4 = jnp.broadcast_to(edge_weight[:, None], (E, 16)).reshape(
        NS, NG, G, CH, 16)
    h2 = _spmm(x2, src4, dst4, w4)
    return _linear(h2, W[_PERM, :], b)


# speedup vs baseline: 1.1517x; 1.0273x over previous
import numpy as np

import jax
import jax.numpy as jnp
from jax import lax
from jax.experimental import pallas as pl
from jax.experimental.pallas import tpu as pltpu
from jax.experimental.pallas import tpu_sc as plsc

N = 10000
E = 320000
D = 128
DH = D // 2
NC = 2
NS = 16
EPT = E // NS
CH = 80
G = 5
NG = EPT // (CH * G)
NP = NG // 2
RPS = 624
TAIL = N - NS * RPS
ZB = 78

_PERM64 = np.concatenate([np.arange(0, 32, 2), np.arange(1, 32, 2),
                          np.arange(32, 64, 2), np.arange(33, 64, 2)])
_PERM = np.concatenate([_PERM64, DH + _PERM64])


def _scale_rows(st, rows_bf, w_blk, b):
    hi_mask = jnp.full((16,), -65536, jnp.int32)

    @plsc.parallel_loop(0, CH, unroll=8)
    def body(r):
        wb = w_blk[b, r]
        for v in range(DH // 32):
            xb = rows_bf[b, r, pl.ds(v * 32, 32)]
            xi = plsc.bitcast(xb, jnp.int32)
            ev = plsc.bitcast(xi << 16, jnp.float32)
            od = plsc.bitcast(xi & hi_mask, jnp.float32)
            st[b, r, pl.ds(v * 32, 16)] = ev * wb
            st[b, r, pl.ds(v * 32 + 16, 16)] = od * wb


def _spmm_body(x_hbm, src_hbm, dst_hbm, w_hbm, out_hbm,
               srcA, dstA, wA, srcB, dstB, wB, rows_bf, st, zbuf, h_sh, sems):
    c = lax.axis_index("c")
    s = lax.axis_index("s")
    gsems = sems[0:G]
    ssems = sems[G:2 * G]
    iAsw, iAd, iBsw, iBd = sems[2 * G:2 * G + 4]

    def load_sw(g, src_buf, w_buf, sem):
        pltpu.async_copy(src_hbm.at[s, g], src_buf, sem)
        pltpu.async_copy(w_hbm.at[s, g], w_buf, sem)

    def wait_sw(src_buf, w_buf, sem):
        pltpu.make_async_copy(src_hbm.at[s, 0], src_buf, sem).wait()
        pltpu.make_async_copy(w_hbm.at[s, 0], w_buf, sem).wait()

    def load_d(g, dst_buf, sem):
        pltpu.async_copy(dst_hbm.at[s, g], dst_buf, sem)

    def wait_d(dst_buf, sem):
        pltpu.make_async_copy(dst_hbm.at[s, 0], dst_buf, sem).wait()

    def gather(src_blk, b):
        pltpu.async_copy(x_hbm.at[c].at[src_blk.at[b]], rows_bf.at[b],
                         gsems[b])

    def wait_gather(src_blk, b):
        pltpu.make_async_copy(x_hbm.at[c].at[src_blk.at[b]], rows_bf.at[b],
                              gsems[b]).wait()

    def scatter(dst_blk, b):
        pltpu.async_copy(st.at[b], h_sh.at[dst_blk.at[b]], ssems[b], add=True)

    def wait_scatter(b):
        pltpu.make_async_copy(st.at[b], h_sh.at[dstA.at[b]], ssems[b]).wait()

    load_sw(0, srcA, wA, iAsw)
    load_d(0, dstA, iAd)
    wait_sw(srcA, wA, iAsw)
    wait_d(dstA, iAd)
    for b in range(G):
        gather(srcA, b)

    zero16 = jnp.zeros((16,), jnp.float32)

    def _zrow(i, carry):
        for v in range(DH // 16):
            zbuf[i, pl.ds(v * 16, 16)] = zero16
        return carry

    lax.fori_loop(0, ZB, _zrow, 0)
    row0 = s * RPS
    for k in range(RPS // ZB):
        pltpu.sync_copy(zbuf, h_sh.at[pl.ds(row0 + k * ZB, ZB)])

    @pl.when(s == NS - 1)
    def _zero_tail():
        pltpu.sync_copy(zbuf.at[pl.ds(0, TAIL)],
                        h_sh.at[pl.ds(NS * RPS, TAIL)])

    plsc.subcore_barrier()

    def _pair(m, carry):
        g0 = 2 * m
        load_sw(g0 + 1, srcB, wB, iBsw)

        @pl.when(m > 0)
        def _dst_a_ready():
            wait_d(dstA, iAd)

        for b in range(G):
            @pl.when(m > 0)
            def _stg_free():
                wait_scatter(b)
            wait_gather(srcA, b)
            _scale_rows(st, rows_bf, wA, b)
            if b == 0:
                wait_sw(srcB, wB, iBsw)
            gather(srcB, b)
            scatter(dstA, b)

        load_d(g0 + 1, dstB, iBd)

        @pl.when(m < NP - 1)
        def _prefetch_sw_a():
            load_sw(g0 + 2, srcA, wA, iAsw)

        for b in range(G):
            wait_scatter(b)
            wait_gather(srcB, b)
            _scale_rows(st, rows_bf, wB, b)
            if b == 0:
                wait_d(dstB, iBd)

            @pl.when(m < NP - 1)
            def _next_gather():
                if b == 0:
                    wait_sw(srcA, wA, iAsw)
                gather(srcA, b)

            scatter(dstB, b)

        @pl.when(m < NP - 1)
        def _prefetch_d_a():
            load_d(g0 + 2, dstA, iAd)

        return carry

    lax.fori_loop(0, NP, _pair, 0)
    for b in range(G):
        wait_scatter(b)
    plsc.subcore_barrier()

    pltpu.sync_copy(h_sh.at[pl.ds(row0, RPS)],
                    out_hbm.at[c].at[pl.ds(row0, RPS)])

    @pl.when(s == NS - 1)
    def _write_tail():
        pltpu.sync_copy(h_sh.at[pl.ds(NS * RPS, TAIL)],
                        out_hbm.at[c].at[pl.ds(NS * RPS, TAIL)])


def _spmm(x2, src4, dst4, w4):
    mesh = plsc.VectorSubcoreMesh(core_axis_name="c", subcore_axis_name="s")

    def body(x_hbm, src_hbm, dst_hbm, w_hbm, out_hbm, srcA, dstA, wA,
             srcB, dstB, wB, rows_bf, st, zbuf, h_sh, *sems):
        _spmm_body(x_hbm, src_hbm, dst_hbm, w_hbm, out_hbm, srcA, dstA, wA,
                   srcB, dstB, wB, rows_bf, st, zbuf, h_sh, list(sems))

    f = pl.kernel(
        body,
        out_type=jax.ShapeDtypeStruct((NC, N, DH), jnp.float32),
        mesh=mesh,
        scratch_types=[
            pltpu.VMEM((G, CH), jnp.int32),
            pltpu.VMEM((G, CH), jnp.int32),
            pltpu.VMEM((G, CH, 16), jnp.float32),
            pltpu.VMEM((G, CH), jnp.int32),
            pltpu.VMEM((G, CH), jnp.int32),
            pltpu.VMEM((G, CH, 16), jnp.float32),
            pltpu.VMEM((G, CH, DH), jnp.bfloat16),
            pltpu.VMEM((G, CH, DH), jnp.float32),
            pltpu.VMEM((ZB, DH), jnp.float32),
            pltpu.VMEM_SHARED((N, DH), jnp.float32),
        ] + [pltpu.SemaphoreType.DMA] * (2 * G + 4),
        compiler_params=pltpu.CompilerParams(use_tc_tiling_on_sc=False,
                                             needs_layout_passes=False),
    )
    return f(x2, src4, dst4, w4)


def _linear_body(h0_ref, h1_ref, w_ref, b_ref, o_ref):
    h = jnp.concatenate([h0_ref[0], h1_ref[0]], axis=1)
    acc = jnp.dot(h, w_ref[...], preferred_element_type=jnp.float32)
    o_ref[...] = jnp.maximum(acc + b_ref[...], 0.0)


def _linear(h2, Wp, b):
    blk = 1000
    grid = (N // blk,)
    return pl.pallas_call(
        _linear_body,
        grid=grid,
        in_specs=[
            pl.BlockSpec((1, blk, DH), lambda i: (0, i, 0)),
            pl.BlockSpec((1, blk, DH), lambda i: (1, i, 0)),
            pl.BlockSpec((D, D), lambda i: (0, 0)),
            pl.BlockSpec((1, D), lambda i: (0, 0)),
        ],
        out_specs=pl.BlockSpec((blk, D), lambda i: (i, 0)),
        out_shape=jax.ShapeDtypeStruct((N, D), jnp.float32),
    )(h2, h2, Wp, b.reshape(1, D))


def kernel(x, edge_index, edge_weight, W, b):
    x2 = jnp.stack([x[:, :DH], x[:, DH:]]).astype(jnp.bfloat16)
    src4 = edge_index[0].reshape(NS, NG, G, CH)
    dst4 = edge_index[1].reshape(NS, NG, G, CH)
    w4 = jnp.broadcast_to(edge_weight[:, None], (E, 16)).reshape(
        NS, NG, G, CH, 16)
    h2 = _spmm(x2, src4, dst4, w4)
    return _linear(h2, W[_PERM, :], b)
